# baseline (speedup 1.0000x reference)
import functools

import jax
import jax.numpy as jnp
from jax.experimental import pallas as pl
from jax.experimental.pallas import tpu as pltpu

NUM_GENES = 18211
INTER_DIM = 1024
LATENT_DIM = 128
BATCH = 64

KBLK = 2048
NK = (NUM_GENES + KBLK - 1) // KBLK


def _mlp_kernel(x_ref, w1_ref, b1_ref, w2_ref, b2_ref, z_ref, acc_ref):
    k = pl.program_id(0)

    @pl.when(k == 0)
    def _init():
        acc_ref[...] = jnp.zeros_like(acc_ref)

    x_blk = x_ref[...]
    w_blk = w1_ref[...]

    @pl.when(k < NK - 1)
    def _accum():
        acc_ref[...] += jnp.dot(
            x_blk.astype(jnp.bfloat16),
            w_blk.astype(jnp.bfloat16),
            preferred_element_type=jnp.float32,
        )

    @pl.when(k == NK - 1)
    def _last():
        base = k * KBLK
        col_ids = jax.lax.broadcasted_iota(jnp.int32, (BATCH, KBLK), 1)
        xm = jnp.where(base + col_ids < NUM_GENES, x_blk, 0.0)
        row_ids = jax.lax.broadcasted_iota(jnp.int32, (KBLK, 1), 0)
        wm = jnp.where(base + row_ids < NUM_GENES, w_blk, 0.0)
        acc = acc_ref[...] + jnp.dot(
            xm.astype(jnp.bfloat16),
            wm.astype(jnp.bfloat16),
            preferred_element_type=jnp.float32,
        )
        h = acc + b1_ref[...]
        h = jnp.where(h > 0, h, 0.25 * h)
        z = jnp.dot(h, w2_ref[...], preferred_element_type=jnp.float32)
        z_ref[...] = z + b2_ref[...]


@functools.partial(jax.jit, static_argnames=())
def kernel(x, W1, b1, W2, b2):
    b1r = b1.reshape(1, INTER_DIM)
    b2r = b2.reshape(1, LATENT_DIM)
    return pl.pallas_call(
        _mlp_kernel,
        grid=(NK,),
        in_specs=[
            pl.BlockSpec((BATCH, KBLK), lambda k: (0, k)),
            pl.BlockSpec((KBLK, INTER_DIM), lambda k: (k, 0)),
            pl.BlockSpec((1, INTER_DIM), lambda k: (0, 0)),
            pl.BlockSpec((INTER_DIM, LATENT_DIM), lambda k: (0, 0)),
            pl.BlockSpec((1, LATENT_DIM), lambda k: (0, 0)),
        ],
        out_specs=pl.BlockSpec((BATCH, LATENT_DIM), lambda k: (0, 0)),
        out_shape=jax.ShapeDtypeStruct((BATCH, LATENT_DIM), jnp.float32),
        scratch_shapes=[pltpu.VMEM((BATCH, INTER_DIM), jnp.float32)],
    )(x, W1, b1r, W2, b2r)
